# Initial kernel scaffold; baseline (speedup 1.0000x reference)
import jax
import jax.numpy as jnp
from jax.experimental import pallas as pl


def kernel(x, edge_index, W1, b1, W2, b2, W3, b3, g1, be1, g2, be2, g3, be3):
    raise NotImplementedError("write your pallas kernel here")


# speedup vs baseline: 5.6902x; 5.6902x over previous
import functools
import jax
import jax.numpy as jnp
from jax import lax
from jax.experimental import pallas as pl
from jax.experimental.pallas import tpu as pltpu
from jax.experimental.pallas import tpu_sc as plsc

NC = 2
NS = 16
BATCH = 128
NA = 10240
EPS = 1e-5


def _make_agg(d, tb, phases):
  mesh = plsc.VectorSubcoreMesh(core_axis_name="c", subcore_axis_name="s")
  rows_per_sub = NA // NS
  tpb = tb // phases

  def body(hs, sidx, didx, zrows, out, acc, sv, dv, r0, r1, gs0, gs1, ss0,
           ss1):
    c = lax.axis_index("c")
    s = lax.axis_index("s")
    t = c * NS + s
    pltpu.sync_copy(zrows, r0)
    for i in range(rows_per_sub // BATCH):
      pltpu.sync_copy(
          r0, acc.at[pl.ds(s * rows_per_sub + i * BATCH, BATCH)])
    plsc.subcore_barrier()

    def g_start(j, buf, sem):
      pltpu.async_copy(hs.at[sv.at[j]], buf, sem)

    def g_wait(j, buf, sem):
      pltpu.make_async_copy(hs.at[sv.at[j]], buf, sem).wait()

    def s_start(j, buf, sem):
      pltpu.async_copy(buf, acc.at[dv.at[j]], sem, add=True)

    def s_wait(j, buf, sem):
      pltpu.make_async_copy(buf, acc.at[dv.at[j]], sem).wait()

    for p in range(phases):
      pltpu.sync_copy(sidx.at[t, pl.ds(p * tpb, tpb)], sv)
      pltpu.sync_copy(didx.at[t, pl.ds(p * tpb, tpb)], dv)
      g_start(0, r0, gs0)
      g_start(1, r1, gs1)

      def loop(g, carry):
        j0 = 2 * g
        j1 = j0 + 1
        g_wait(j0, r0, gs0)
        s_start(j0, r0, ss0)
        g_wait(j1, r1, gs1)
        s_start(j1, r1, ss1)
        s_wait(j0, r0, ss0)

        @pl.when(j0 + 2 < tpb)
        def _():
          g_start(j0 + 2, r0, gs0)

        s_wait(j1, r1, ss1)

        @pl.when(j1 + 2 < tpb)
        def _():
          g_start(j1 + 2, r1, gs1)

        return carry

      lax.fori_loop(0, tpb // 2, loop, 0)

    plsc.subcore_barrier()
    for i in range(rows_per_sub // BATCH):
      off = s * rows_per_sub + i * BATCH
      pltpu.sync_copy(acc.at[pl.ds(off, BATCH)], r0)
      pltpu.sync_copy(r0, out.at[c, pl.ds(off, BATCH)])

  return pl.kernel(
      body,
      out_type=jax.ShapeDtypeStruct((NC, NA, d), jnp.float32),
      mesh=mesh,
      scratch_types=[
          pltpu.VMEM_SHARED((NA, d), jnp.float32),
          pltpu.VMEM((tb // phases, BATCH), jnp.int32),
          pltpu.VMEM((tb // phases, BATCH), jnp.int32),
          pltpu.VMEM((BATCH, d), jnp.float32),
          pltpu.VMEM((BATCH, d), jnp.float32),
          pltpu.SemaphoreType.DMA,
          pltpu.SemaphoreType.DMA,
          pltpu.SemaphoreType.DMA,
          pltpu.SemaphoreType.DMA,
      ],
  )


def _make_deg(tb):
  mesh = plsc.VectorSubcoreMesh(core_axis_name="c", subcore_axis_name="s")
  rows_per_sub = NA // NS
  d = 128

  def body(didx, ones, zrows, out, acc, dv, ob, zb, ssem):
    c = lax.axis_index("c")
    s = lax.axis_index("s")
    t = c * NS + s
    pltpu.sync_copy(didx.at[t], dv)
    pltpu.sync_copy(ones, ob)
    pltpu.sync_copy(zrows, zb)
    for i in range(rows_per_sub // BATCH):
      pltpu.sync_copy(
          zb, acc.at[pl.ds(s * rows_per_sub + i * BATCH, BATCH)])
    plsc.subcore_barrier()

    def s_start(j):
      pltpu.async_copy(ob, acc.at[dv.at[j]], ssem, add=True)

    def s_wait(j):
      pltpu.make_async_copy(ob, acc.at[dv.at[j]], ssem).wait()

    def loop(g, carry):
      j0 = 2 * g
      s_start(j0)
      s_start(j0 + 1)
      s_wait(j0)
      s_wait(j0 + 1)
      return carry

    lax.fori_loop(0, tb // 2, loop, 0)
    plsc.subcore_barrier()
    for i in range(rows_per_sub // BATCH):
      off = s * rows_per_sub + i * BATCH
      pltpu.sync_copy(acc.at[pl.ds(off, BATCH)], zb)
      pltpu.sync_copy(zb, out.at[c, pl.ds(off, BATCH)])

  return pl.kernel(
      body,
      out_type=jax.ShapeDtypeStruct((NC, NA, d), jnp.float32),
      mesh=mesh,
      scratch_types=[
          pltpu.VMEM_SHARED((NA, d), jnp.float32),
          pltpu.VMEM((tb, BATCH), jnp.int32),
          pltpu.VMEM((BATCH, d), jnp.float32),
          pltpu.VMEM((BATCH, d), jnp.float32),
          pltpu.SemaphoreType.DMA,
      ],
  )


def _tc_first(n):
  def body(x_ref, w_ref, degp_ref, hs_ref, dinv_ref):
    deg = degp_ref[0, :n, 0:1] + degp_ref[1, :n, 0:1] + 1.0
    dinv = lax.rsqrt(deg)
    h = jnp.dot(x_ref[...], w_ref[...], preferred_element_type=jnp.float32)
    hs_ref[...] = h * dinv
    dinv_ref[...] = dinv

  return body


def _tc_mid(n):
  def body(aggp_ref, hs_ref, dinv_ref, b_ref, g_ref, be_ref, w_ref, out_ref):
    dinv = dinv_ref[...]
    z = (aggp_ref[0, :n] + aggp_ref[1, :n] + hs_ref[...]) * dinv + b_ref[...]
    m = jnp.mean(z, axis=0)
    v = jnp.mean((z - m) * (z - m), axis=0)
    zn = (z - m) * lax.rsqrt(v + EPS) * g_ref[...] + be_ref[...]
    r = jnp.maximum(zn, 0.0)
    out_ref[...] = (
        jnp.dot(r, w_ref[...], preferred_element_type=jnp.float32) * dinv)

  return body


def _tc_final(n, c_dim):
  def body(aggp_ref, hs_ref, dinv_ref, b_ref, g_ref, be_ref, out_ref):
    z = (aggp_ref[0, :n] + aggp_ref[1, :n] + hs_ref[...]) * dinv_ref[...]
    z = z[:, :c_dim] + b_ref[...]
    m = jnp.mean(z, axis=-1, keepdims=True)
    v = jnp.mean((z - m) * (z - m), axis=-1, keepdims=True)
    ln = (z - m) * lax.rsqrt(v + EPS) * g_ref[...] + be_ref[...]
    mx = jnp.max(ln, axis=-1, keepdims=True)
    lse = mx + jnp.log(jnp.sum(jnp.exp(ln - mx), axis=-1, keepdims=True))
    out_ref[...] = ln - lse

  return body


def kernel(x, edge_index, W1, b1, W2, b2, W3, b3, g1, be1, g2, be2, g3, be3):
  n, _ = x.shape
  e = edge_index.shape[1]
  h_dim = W1.shape[1]
  c_dim = W3.shape[1]
  c_pad = 128

  tb = -(-e // (NC * NS * BATCH))
  tb = -(-tb // 16) * 16
  ep = NC * NS * tb * BATCH
  src = jnp.concatenate([edge_index[0], jnp.zeros((ep - e,), jnp.int32)])
  dst = jnp.concatenate(
      [edge_index[1], jnp.full((ep - e,), n + 8, jnp.int32)])
  srcp = src.reshape(NC * NS, tb, BATCH)
  dstp = dst.reshape(NC * NS, tb, BATCH)

  ones8 = jnp.ones((BATCH, 128), jnp.float32)
  zdeg = jnp.zeros((BATCH, 128), jnp.float32)
  zh = jnp.zeros((BATCH, h_dim), jnp.float32)
  zc = jnp.zeros((BATCH, c_pad), jnp.float32)

  W3p = jnp.pad(W3, ((0, 0), (0, c_pad - c_dim)))

  degp = _make_deg(tb)(dstp, ones8, zdeg)

  agg_h = _make_agg(h_dim, tb, 2)
  assert c_pad == h_dim
  agg_c = agg_h

  hs1, dinv = pl.pallas_call(
      _tc_first(n),
      out_shape=(
          jax.ShapeDtypeStruct((n, h_dim), jnp.float32),
          jax.ShapeDtypeStruct((n, 1), jnp.float32),
      ),
  )(x, W1, degp)

  agg1 = agg_h(hs1, srcp, dstp, zh)
  hs2 = pl.pallas_call(
      _tc_mid(n),
      out_shape=jax.ShapeDtypeStruct((n, h_dim), jnp.float32),
  )(agg1, hs1, dinv, b1, g1, be1, W2)

  agg2 = agg_h(hs2, srcp, dstp, zh)
  hs3 = pl.pallas_call(
      _tc_mid(n),
      out_shape=jax.ShapeDtypeStruct((n, c_pad), jnp.float32),
  )(agg2, hs2, dinv, b2, g2, be2, W3p)

  agg3 = agg_c(hs3, srcp, dstp, zc)
  out = pl.pallas_call(
      _tc_final(n, c_dim),
      out_shape=jax.ShapeDtypeStruct((n, c_dim), jnp.float32),
  )(agg3, hs3, dinv, b3, g3, be3)
  return out


# speedup vs baseline: 7.2384x; 1.2721x over previous
import functools
import jax
import jax.numpy as jnp
from jax import lax
from jax.experimental import pallas as pl
from jax.experimental.pallas import tpu as pltpu
from jax.experimental.pallas import tpu_sc as plsc

NC = 2
NS = 16
BATCH = 128
NA = 10112
CNT0 = 128
CNT1 = 32
EPS = 1e-5

_ROWS_PER_SUB = NA // NS
_CHUNKS = [128, 128, 128, 128, _ROWS_PER_SUB - 512]


def _make_agg(d):
  mesh = plsc.VectorSubcoreMesh(core_axis_name="c", subcore_axis_name="s")
  nb0 = NS * CNT0

  def body(hs, sidx, didx, zrows, out, acc, sv, dv, r0, r1, gs0, gs1, ss0,
           ss1):
    c = lax.axis_index("c")
    s = lax.axis_index("s")
    pltpu.sync_copy(zrows, r0)
    pos = 0
    for sz in _CHUNKS:
      pltpu.sync_copy(
          r0.at[pl.ds(0, sz)],
          acc.at[pl.ds(s * _ROWS_PER_SUB + pos, sz)])
      pos += sz
    plsc.subcore_barrier()

    def g_start(j, buf, sem):
      pltpu.async_copy(hs.at[sv.at[j]], buf, sem)

    def g_wait(j, buf, sem):
      pltpu.make_async_copy(hs.at[sv.at[j]], buf, sem).wait()

    def s_start(j, buf, sem):
      pltpu.async_copy(buf, acc.at[dv.at[j]], sem, add=True)

    def s_wait(j, buf, sem):
      pltpu.make_async_copy(buf, acc.at[dv.at[j]], sem).wait()

    def pipeline(base, cnt, phases):
      tpb = cnt // phases
      for p in range(phases):
        off = base + p * tpb
        pltpu.sync_copy(sidx.at[pl.ds(off, tpb)], sv.at[pl.ds(0, tpb)])
        pltpu.sync_copy(didx.at[pl.ds(off, tpb)], dv.at[pl.ds(0, tpb)])
        g_start(0, r0, gs0)
        g_start(1, r1, gs1)

        def loop(g, carry):
          j0 = 2 * g
          j1 = j0 + 1
          g_wait(j0, r0, gs0)
          s_start(j0, r0, ss0)
          g_wait(j1, r1, gs1)
          s_start(j1, r1, ss1)
          s_wait(j0, r0, ss0)

          @pl.when(j0 + 2 < tpb)
          def _():
            g_start(j0 + 2, r0, gs0)

          s_wait(j1, r1, ss1)

          @pl.when(j1 + 2 < tpb)
          def _():
            g_start(j1 + 2, r1, gs1)

          return carry

        lax.fori_loop(0, tpb // 2, loop, 0)

    @pl.when(c == 0)
    def _():
      pipeline(s * CNT0, CNT0, max(1, CNT0 // 64))

    @pl.when(c == 1)
    def _():
      pipeline(nb0 + s * CNT1, CNT1, max(1, CNT1 // 64))

    plsc.subcore_barrier()
    pos = 0
    for sz in _CHUNKS:
      off = s * _ROWS_PER_SUB + pos
      pltpu.sync_copy(acc.at[pl.ds(off, sz)], r0.at[pl.ds(0, sz)])
      pltpu.sync_copy(r0.at[pl.ds(0, sz)], out.at[c, pl.ds(off, sz)])
      pos += sz

  sbuf = max(CNT0 // max(1, CNT0 // 64), CNT1 // max(1, CNT1 // 64))
  return pl.kernel(
      body,
      out_type=jax.ShapeDtypeStruct((NC, NA, d), jnp.float32),
      mesh=mesh,
      scratch_types=[
          pltpu.VMEM_SHARED((NA, d), jnp.float32),
          pltpu.VMEM((sbuf, BATCH), jnp.int32),
          pltpu.VMEM((sbuf, BATCH), jnp.int32),
          pltpu.VMEM((BATCH, d), jnp.float32),
          pltpu.VMEM((BATCH, d), jnp.float32),
          pltpu.SemaphoreType.DMA,
          pltpu.SemaphoreType.DMA,
          pltpu.SemaphoreType.DMA,
          pltpu.SemaphoreType.DMA,
      ],
  )


def _make_deg(tb):
  mesh = plsc.VectorSubcoreMesh(core_axis_name="c", subcore_axis_name="s")
  d = 128

  def body(didx, ones, zrows, out, acc, dv, ob, zb, ssem):
    c = lax.axis_index("c")
    s = lax.axis_index("s")
    t = c * NS + s
    pltpu.sync_copy(didx.at[pl.ds(t * tb, tb)], dv)
    pltpu.sync_copy(ones, ob)
    pltpu.sync_copy(zrows, zb)
    pos = 0
    for sz in _CHUNKS:
      pltpu.sync_copy(
          zb.at[pl.ds(0, sz)],
          acc.at[pl.ds(s * _ROWS_PER_SUB + pos, sz)])
      pos += sz
    plsc.subcore_barrier()

    def s_start(j):
      pltpu.async_copy(ob, acc.at[dv.at[j]], ssem, add=True)

    def s_wait(j):
      pltpu.make_async_copy(ob, acc.at[dv.at[j]], ssem).wait()

    def loop(g, carry):
      j0 = 2 * g
      s_start(j0)
      s_start(j0 + 1)
      s_wait(j0)
      s_wait(j0 + 1)
      return carry

    lax.fori_loop(0, tb // 2, loop, 0)
    plsc.subcore_barrier()
    pos = 0
    for sz in _CHUNKS:
      off = s * _ROWS_PER_SUB + pos
      pltpu.sync_copy(acc.at[pl.ds(off, sz)], zb.at[pl.ds(0, sz)])
      pltpu.sync_copy(zb.at[pl.ds(0, sz)], out.at[c, pl.ds(off, sz)])
      pos += sz

  return pl.kernel(
      body,
      out_type=jax.ShapeDtypeStruct((NC, NA, d), jnp.float32),
      mesh=mesh,
      scratch_types=[
          pltpu.VMEM_SHARED((NA, d), jnp.float32),
          pltpu.VMEM((tb, BATCH), jnp.int32),
          pltpu.VMEM((BATCH, d), jnp.float32),
          pltpu.VMEM((BATCH, d), jnp.float32),
          pltpu.SemaphoreType.DMA,
      ],
  )


def _tc_first(n):
  def body(x_ref, w_ref, degp_ref, hs_ref, dinv_ref):
    deg = degp_ref[0, :n, 0:1] + degp_ref[1, :n, 0:1] + 1.0
    dinv = lax.rsqrt(deg)
    h = jnp.dot(x_ref[...], w_ref[...], preferred_element_type=jnp.float32)
    hs_ref[...] = h * dinv
    dinv_ref[...] = dinv

  return body


def _tc_mid(n):
  def body(aggp_ref, hs_ref, dinv_ref, b_ref, g_ref, be_ref, w_ref, out_ref):
    dinv = dinv_ref[...]
    z = (aggp_ref[0, :n] + aggp_ref[1, :n] + hs_ref[...]) * dinv + b_ref[...]
    m = jnp.mean(z, axis=0)
    v = jnp.mean((z - m) * (z - m), axis=0)
    zn = (z - m) * lax.rsqrt(v + EPS) * g_ref[...] + be_ref[...]
    r = jnp.maximum(zn, 0.0)
    out_ref[...] = (
        jnp.dot(r, w_ref[...], preferred_element_type=jnp.float32) * dinv)

  return body


def _tc_final(n, c_dim):
  def body(aggp_ref, hs_ref, dinv_ref, b_ref, g_ref, be_ref, out_ref):
    z = (aggp_ref[0, :n] + aggp_ref[1, :n] + hs_ref[...]) * dinv_ref[...]
    z = z[:, :c_dim] + b_ref[...]
    m = jnp.mean(z, axis=-1, keepdims=True)
    v = jnp.mean((z - m) * (z - m), axis=-1, keepdims=True)
    ln = (z - m) * lax.rsqrt(v + EPS) * g_ref[...] + be_ref[...]
    mx = jnp.max(ln, axis=-1, keepdims=True)
    lse = mx + jnp.log(jnp.sum(jnp.exp(ln - mx), axis=-1, keepdims=True))
    out_ref[...] = ln - lse

  return body


def kernel(x, edge_index, W1, b1, W2, b2, W3, b3, g1, be1, g2, be2, g3, be3):
  n, _ = x.shape
  e = edge_index.shape[1]
  h_dim = W1.shape[1]
  c_dim = W3.shape[1]
  c_pad = 128

  tbt = NS * (CNT0 + CNT1)
  tb_even = tbt // (NC * NS)
  ep = tbt * BATCH
  src = jnp.concatenate([edge_index[0], jnp.zeros((ep - e,), jnp.int32)])
  dst = jnp.concatenate(
      [edge_index[1], jnp.full((ep - e,), n + 8, jnp.int32)])
  srcp = src.reshape(tbt, BATCH)
  dstp = dst.reshape(tbt, BATCH)

  ones = jnp.ones((BATCH, 128), jnp.float32)
  zrows = jnp.zeros((BATCH, 128), jnp.float32)

  W3p = jnp.pad(W3, ((0, 0), (0, c_pad - c_dim)))

  degp = _make_deg(tb_even)(dstp, ones, zrows)

  agg = _make_agg(h_dim)
  assert c_pad == h_dim

  hs1, dinv = pl.pallas_call(
      _tc_first(n),
      out_shape=(
          jax.ShapeDtypeStruct((n, h_dim), jnp.float32),
          jax.ShapeDtypeStruct((n, 1), jnp.float32),
      ),
  )(x, W1, degp)

  agg1 = agg(hs1, srcp, dstp, zrows)
  hs2 = pl.pallas_call(
      _tc_mid(n),
      out_shape=jax.ShapeDtypeStruct((n, h_dim), jnp.float32),
  )(agg1, hs1, dinv, b1, g1, be1, W2)

  agg2 = agg(hs2, srcp, dstp, zrows)
  hs3 = pl.pallas_call(
      _tc_mid(n),
      out_shape=jax.ShapeDtypeStruct((n, c_pad), jnp.float32),
  )(agg2, hs2, dinv, b2, g2, be2, W3p)

  agg3 = agg(hs3, srcp, dstp, zrows)
  out = pl.pallas_call(
      _tc_final(n, c_dim),
      out_shape=jax.ShapeDtypeStruct((n, c_dim), jnp.float32),
  )(agg3, hs3, dinv, b3, g3, be3)
  return out
